# baseline (speedup 1.0000x reference)
import functools

import jax
import jax.numpy as jnp
from jax import lax
from jax.experimental import pallas as pl
from jax.experimental.pallas import tpu as pltpu
from jax.experimental.pallas import tpu_sc as plsc

D = 32
NI = 16384
NJ = 200
NW = 32
UNIT = 128
UNITS = NJ * (NI // UNIT)
U_PER_W = UNITS // NW
SUPER = 50
NSC = U_PER_W // SUPER
NG = 4
DT = 4
EMB_DIM_PAD_ROW = 32


@functools.lru_cache(maxsize=None)
def _build():
    mesh = plsc.VectorSubcoreMesh(core_axis_name="c", subcore_axis_name="s")

    @functools.partial(
        pl.kernel,
        mesh=mesh,
        out_type=jax.ShapeDtypeStruct((NJ, DT, NI // UNIT, 8, UNIT),
                                      jnp.float32),
        scratch_types=[
            pltpu.VMEM((SUPER * UNIT,), jnp.int32),
            pltpu.VMEM((SUPER * UNIT,), jnp.int32),
            pltpu.VMEM((UNIT, D), jnp.float32),
            pltpu.VMEM((UNIT, D), jnp.float32),
            pltpu.VMEM((UNIT, D), jnp.float32),
            pltpu.VMEM((UNIT, D), jnp.float32),
            pltpu.VMEM((UNIT,), jnp.int32),
            pltpu.VMEM((UNIT,), jnp.int32),
            pltpu.VMEM((UNIT,), jnp.int32),
            pltpu.VMEM((UNIT,), jnp.int32),
            pltpu.VMEM((D, UNIT + 1), jnp.float32),
            pltpu.VMEM((D, UNIT + 1), jnp.float32),
            pltpu.SemaphoreType.DMA,
            pltpu.SemaphoreType.DMA,
            pltpu.SemaphoreType.DMA,
            pltpu.SemaphoreType.DMA,
            pltpu.SemaphoreType.DMA,
            pltpu.SemaphoreType.DMA,
            pltpu.SemaphoreType.DMA,
            pltpu.SemaphoreType.DMA,
        ],
        compiler_params=pltpu.CompilerParams(use_tc_tiling_on_sc=False,
                                             needs_layout_passes=False),
    )
    def emb_lookup(xt_hbm, tbl_hbm, out_hbm,
                   idxA, idxB, rows0, rows1, rows2, rows3,
                   sidx0, sidx1, sidx2, sidx3, slab0, slab1,
                   isemA, isemB, gsem0, gsem1, gsem2, gsem3, ssem0, ssem1):
        sidx = (sidx0, sidx1, sidx2, sidx3)
        idx_v = (idxA, idxB)
        isem = (isemA, isemB)
        rows = (rows0, rows1, rows2, rows3)
        gsem = (gsem0, gsem1, gsem2, gsem3)
        slab = (slab0, slab1)
        ssem = (ssem0, ssem1)

        wid = lax.axis_index("s") * 2 + lax.axis_index("c")
        u_base = wid * U_PER_W

        def idx_load(s, p):
            off = (u_base + s * SUPER) * UNIT
            return pltpu.make_async_copy(
                xt_hbm.at[pl.ds(off, SUPER * UNIT)], idx_v[p], isem[p])

        def idx_wait(p):
            pltpu.make_async_copy(
                xt_hbm.at[pl.ds(0, SUPER * UNIT)], idx_v[p], isem[p]).wait()

        def gather_start(u, bb, p):
            off = (u % SUPER) * UNIT
            for k in range(UNIT // 16):
                sidx[bb][pl.ds(16 * k, 16)] = (
                    idx_v[p][pl.ds(off + 16 * k, 16)] << 2)
            pltpu.make_async_copy(
                tbl_hbm.at[sidx[bb]], rows[bb], gsem[bb]).start()

        def gather_wait(bb):
            pltpu.make_async_copy(
                tbl_hbm.at[sidx[bb]], rows[bb], gsem[bb]).wait()

        def store_start(u, tb):
            gu = u_base + u
            o = gu >> 3
            r = gu & 7
            j = ((o >> 7) << 3) | r
            ti = o & 127
            for td in range(DT):
                pltpu.make_async_copy(
                    slab[tb].at[pl.ds(8 * td, 8), pl.ds(0, UNIT)],
                    out_hbm.at[j, td, ti], ssem[tb]).start()

        def store_wait(tb):
            for td in range(DT):
                pltpu.make_async_copy(
                    slab[tb].at[pl.ds(8 * td, 8), pl.ds(0, UNIT)],
                    out_hbm.at[0, 0, 0], ssem[tb]).wait()

        viota = lax.iota(jnp.int32, 16)
        didx = [viota + (16 * h) for h in range(D // 16)]
        czero = viota * 0

        def transpose(bb, tb):
            r_ref = rows[bb]
            s_ref = slab[tb]

            @plsc.parallel_loop(0, UNIT, unroll=4)
            def tbody(c):
                cvec = czero + c
                for h in range(D // 16):
                    v = r_ref[c, pl.ds(16 * h, 16)]
                    plsc.store_scatter(s_ref, [didx[h], cvec], v)

        def issue_next_gather(u, bb):
            nu = u + NG
            p_next = (nu // SUPER) % 2

            @pl.when(nu % SUPER == 0)
            def _():
                @pl.when(p_next == 0)
                def _():
                    idx_wait(0)

                @pl.when(p_next == 1)
                def _():
                    idx_wait(1)

            @pl.when(p_next == 0)
            def _():
                gather_start(nu, bb, 0)

            @pl.when(p_next == 1)
            def _():
                gather_start(nu, bb, 1)

        idx_load(0, 0).start()
        idx_load(1, 1).start()
        idx_wait(0)
        for bb in range(NG):
            gather_start(bb, bb, 0)
        for bb in range(NG):
            gather_wait(bb)
            tb = bb % 2
            if bb >= 2:
                store_wait(tb)
            transpose(bb, tb)
            store_start(bb, tb)
            issue_next_gather(bb, bb)

        def body(g, carry):
            for bb in range(NG):
                u = g * NG + bb
                tb = bb % 2
                s = u // SUPER

                @pl.when((lax.rem(u, SUPER) == 0) & (u + SUPER < U_PER_W))
                def _():
                    sp = (s + 1) % 2

                    @pl.when(sp == 0)
                    def _():
                        idx_load(s + 1, 0).start()

                    @pl.when(sp == 1)
                    def _():
                        idx_load(s + 1, 1).start()

                gather_wait(bb)
                store_wait(tb)
                transpose(bb, tb)
                store_start(u, tb)

                @pl.when(u + NG < U_PER_W)
                def _():
                    issue_next_gather(u, bb)
            return carry

        lax.fori_loop(1, U_PER_W // NG, body, 0, unroll=False)

        store_wait(0)
        store_wait(1)

    return emb_lookup


def kernel(x, emb_weight):
    xp = x.astype(jnp.int32).reshape(128, 128, 25, 8).transpose(2, 0, 3, 1)
    tbl4 = jnp.broadcast_to(emb_weight[:, None, :],
                            (emb_weight.shape[0], 4, D)).reshape(-1, D)
    out5 = _build()(xp.reshape(-1), tbl4)
    o1 = out5.transpose(0, 1, 3, 2, 4)
    o2 = o1.reshape(NJ, D, NI)
    return o2.transpose(2, 0, 1)


# speedup vs baseline: 3.3634x; 3.3634x over previous
import functools

import jax
import jax.numpy as jnp
from jax import lax
from jax.experimental import pallas as pl
from jax.experimental.pallas import tpu as pltpu
from jax.experimental.pallas import tpu_sc as plsc

D = 32
NI = 16384
NJ = 200
NW = 32
UNIT = 128
UNITS = NJ * (NI // UNIT)
U_PER_W = UNITS // NW
SUPER = 50
NSC = U_PER_W // SUPER
NG = 8
DT = 4
EMB_DIM_PAD_ROW = 32


@functools.lru_cache(maxsize=None)
def _build():
    mesh = plsc.VectorSubcoreMesh(core_axis_name="c", subcore_axis_name="s")

    @functools.partial(
        pl.kernel,
        mesh=mesh,
        out_type=jax.ShapeDtypeStruct((NJ, DT, NI // UNIT, 8, UNIT),
                                      jnp.float32),
        scratch_types=[
            pltpu.VMEM((SUPER * UNIT,), jnp.int32),
            pltpu.VMEM((SUPER * UNIT,), jnp.int32),
            pltpu.VMEM((UNIT, D), jnp.float32),
            pltpu.VMEM((UNIT, D), jnp.float32),
            pltpu.VMEM((UNIT, D), jnp.float32),
            pltpu.VMEM((UNIT, D), jnp.float32),
            pltpu.VMEM((UNIT, D), jnp.float32),
            pltpu.VMEM((UNIT, D), jnp.float32),
            pltpu.VMEM((UNIT, D), jnp.float32),
            pltpu.VMEM((UNIT, D), jnp.float32),
            pltpu.VMEM((UNIT,), jnp.int32),
            pltpu.VMEM((UNIT,), jnp.int32),
            pltpu.VMEM((UNIT,), jnp.int32),
            pltpu.VMEM((UNIT,), jnp.int32),
            pltpu.VMEM((UNIT,), jnp.int32),
            pltpu.VMEM((UNIT,), jnp.int32),
            pltpu.VMEM((UNIT,), jnp.int32),
            pltpu.VMEM((UNIT,), jnp.int32),
            pltpu.VMEM((D, UNIT + 1), jnp.float32),
            pltpu.VMEM((D, UNIT + 1), jnp.float32),
            pltpu.SemaphoreType.DMA,
            pltpu.SemaphoreType.DMA,
            pltpu.SemaphoreType.DMA,
            pltpu.SemaphoreType.DMA,
            pltpu.SemaphoreType.DMA,
            pltpu.SemaphoreType.DMA,
            pltpu.SemaphoreType.DMA,
            pltpu.SemaphoreType.DMA,
            pltpu.SemaphoreType.DMA,
            pltpu.SemaphoreType.DMA,
            pltpu.SemaphoreType.DMA,
            pltpu.SemaphoreType.DMA,
        ],
        compiler_params=pltpu.CompilerParams(use_tc_tiling_on_sc=False,
                                             needs_layout_passes=False),
    )
    def emb_lookup(xt_hbm, tbl_hbm, out_hbm,
                   idxA, idxB, rows0, rows1, rows2, rows3, rows4, rows5,
                   rows6, rows7, sidx0, sidx1, sidx2, sidx3, sidx4, sidx5,
                   sidx6, sidx7, slab0, slab1,
                   isemA, isemB, gsem0, gsem1, gsem2, gsem3, gsem4, gsem5,
                   gsem6, gsem7, ssem0, ssem1):
        sidx = (sidx0, sidx1, sidx2, sidx3, sidx4, sidx5, sidx6, sidx7)
        idx_v = (idxA, idxB)
        isem = (isemA, isemB)
        rows = (rows0, rows1, rows2, rows3, rows4, rows5, rows6, rows7)
        gsem = (gsem0, gsem1, gsem2, gsem3, gsem4, gsem5, gsem6, gsem7)
        slab = (slab0, slab1)
        ssem = (ssem0, ssem1)

        wid = lax.axis_index("s") * 2 + lax.axis_index("c")
        u_base = wid * U_PER_W

        def idx_load(s, p):
            off = (u_base + s * SUPER) * UNIT
            return pltpu.make_async_copy(
                xt_hbm.at[pl.ds(off, SUPER * UNIT)], idx_v[p], isem[p])

        def idx_wait(p):
            pltpu.make_async_copy(
                xt_hbm.at[pl.ds(0, SUPER * UNIT)], idx_v[p], isem[p]).wait()

        def gather_start(u, bb, p):
            off = (u % SUPER) * UNIT
            for k in range(UNIT // 16):
                sidx[bb][pl.ds(16 * k, 16)] = (
                    idx_v[p][pl.ds(off + 16 * k, 16)] << 2)
            pltpu.make_async_copy(
                tbl_hbm.at[sidx[bb]], rows[bb], gsem[bb]).start()

        def gather_wait(bb):
            pltpu.make_async_copy(
                tbl_hbm.at[sidx[bb]], rows[bb], gsem[bb]).wait()

        def store_start(u, tb):
            gu = u_base + u
            o = gu >> 3
            r = gu & 7
            j = ((o >> 7) << 3) | r
            ti = o & 127
            for td in range(DT):
                pltpu.make_async_copy(
                    slab[tb].at[pl.ds(8 * td, 8), pl.ds(0, UNIT)],
                    out_hbm.at[j, td, ti], ssem[tb]).start()

        def store_wait(tb):
            for td in range(DT):
                pltpu.make_async_copy(
                    slab[tb].at[pl.ds(8 * td, 8), pl.ds(0, UNIT)],
                    out_hbm.at[0, 0, 0], ssem[tb]).wait()

        viota = lax.iota(jnp.int32, 16)
        didx = [viota + (16 * h) for h in range(D // 16)]
        czero = viota * 0

        def transpose(bb, tb):
            r_ref = rows[bb]
            s_ref = slab[tb]

            @plsc.parallel_loop(0, UNIT, unroll=4)
            def tbody(c):
                cvec = czero + c
                for h in range(D // 16):
                    v = r_ref[c, pl.ds(16 * h, 16)]
                    plsc.store_scatter(s_ref, [didx[h], cvec], v)

        def issue_next_gather(u, bb):
            nu = u + NG
            p_next = (nu // SUPER) % 2

            @pl.when(nu % SUPER == 0)
            def _():
                @pl.when(p_next == 0)
                def _():
                    idx_wait(0)

                @pl.when(p_next == 1)
                def _():
                    idx_wait(1)

            @pl.when(p_next == 0)
            def _():
                gather_start(nu, bb, 0)

            @pl.when(p_next == 1)
            def _():
                gather_start(nu, bb, 1)

        idx_load(0, 0).start()
        idx_load(1, 1).start()
        idx_wait(0)
        for bb in range(NG):
            gather_start(bb, bb, 0)
        for bb in range(NG):
            gather_wait(bb)
            tb = bb % 2
            if bb >= 2:
                store_wait(tb)
            transpose(bb, tb)
            store_start(bb, tb)
            issue_next_gather(bb, bb)

        def body(g, carry):
            for bb in range(NG):
                u = g * NG + bb
                tb = bb % 2
                s = u // SUPER

                @pl.when((lax.rem(u, SUPER) == 0) & (u + SUPER < U_PER_W))
                def _():
                    sp = (s + 1) % 2

                    @pl.when(sp == 0)
                    def _():
                        idx_load(s + 1, 0).start()

                    @pl.when(sp == 1)
                    def _():
                        idx_load(s + 1, 1).start()

                gather_wait(bb)
                store_wait(tb)
                transpose(bb, tb)
                store_start(u, tb)

                @pl.when(u + NG < U_PER_W)
                def _():
                    issue_next_gather(u, bb)
            return carry

        lax.fori_loop(1, U_PER_W // NG, body, 0, unroll=False)

        store_wait(0)
        store_wait(1)

    return emb_lookup


def kernel(x, emb_weight):
    xp = x.astype(jnp.int32).reshape(128, 128, 25, 8).transpose(2, 0, 3, 1)
    tbl4 = jnp.pad(emb_weight, ((0, 0), (0, 96))).reshape(-1, EMB_DIM_PAD_ROW)
    out5 = _build()(xp.reshape(-1), tbl4)
    o1 = out5.transpose(0, 1, 3, 2, 4)
    o2 = o1.reshape(NJ, D, NI)
    return o2.transpose(2, 0, 1)
